# baseline (speedup 1.0000x reference)
import functools

import numpy as np
import jax
import jax.numpy as jnp
from jax import lax
from jax.experimental import pallas as pl
from jax.experimental.pallas import tpu as pltpu

HW = 56
NPX = HW * HW
CO = 1792
NCEN = 3136
PXT = 224
NT = NPX // PXT
RB = 56


def _pool_mat(n: int) -> np.ndarray:
    m = np.zeros((n, n), np.float32)
    for i in range(n):
        for j in (i - 1, i, i + 1):
            if 0 <= j < n:
                m[i, j] = 1.0 / 3.0
    return m


def _resize_mat(n_out: int, n_in: int) -> np.ndarray:
    m = np.zeros((n_out, n_in), np.float32)
    scale = n_in / n_out
    for i in range(n_out):
        c = (i + 0.5) * scale - 0.5
        j0 = int(np.floor(c))
        f = c - j0
        m[i, min(max(j0, 0), n_in - 1)] += 1.0 - f
        m[i, min(max(j0 + 1, 0), n_in - 1)] += f
    return m


_A1 = (_resize_mat(56, 28) @ _pool_mat(28)).astype(np.float32)
_A2 = (_resize_mat(56, 14) @ _pool_mat(14)).astype(np.float32)
_S2 = np.einsum("yv,xu->xyvu", _A2, _A2).reshape(NPX, 196).astype(np.float32)

_xx = (np.arange(56, dtype=np.float32) / 55.0) * 2.0 - 1.0
_CM = np.zeros((NPX, 8), np.float32)
_CM[:, 0] = np.repeat(_xx, 56)
_CM[:, 1] = np.tile(_xx, 56)
_CM[:, 2] = 1.0


def _mix_p2_kern(p2_ref, w2t_ref, out_ref):
    out_ref[0] = lax.dot_general(p2_ref[0], w2t_ref[...],
                                 (((0,), (0,)), ((), ())),
                                 preferred_element_type=jnp.float32)


def _e2_kern(s2_ref, m2_ref, cm_ref, wc_ref, e0_ref, e1_ref, out_ref):
    e2 = jnp.dot(s2_ref[...], m2_ref[0], preferred_element_type=jnp.float32)
    e2 = e2 + jnp.dot(cm_ref[...], wc_ref[...], preferred_element_type=jnp.float32)
    out_ref[0] = e2 + e0_ref[0] + e1_ref[0]


def _cc_kern(cen_ref, out_ref):
    i = pl.program_id(0)
    part = jnp.broadcast_to(
        jnp.sum(cen_ref[...] * cen_ref[...], axis=0, keepdims=True),
        out_ref.shape)

    @pl.when(i == 0)
    def _():
        out_ref[...] = part

    @pl.when(i > 0)
    def _():
        out_ref[...] = out_ref[...] + part


def _mix_p1_kern(p1_ref, w1t_ref, out_ref):
    out_ref[0] = lax.dot_general(p1_ref[0], w1t_ref[...],
                                 (((0,), (0,)), ((), ())),
                                 preferred_element_type=jnp.float32)


def _hstep_kern(a_ref, m1_ref, out_ref):
    out_ref[0] = jnp.dot(a_ref[...], m1_ref[0], preferred_element_type=jnp.float32)


def _wstep_kern(a_ref, z_ref, out_ref):
    for j in range(z_ref.shape[1]):
        out_ref[0, :, j, :] = jnp.dot(a_ref[...], z_ref[0, j],
                                      preferred_element_type=jnp.float32)


def _mix_p0_kern(pooled_ref, w0t_ref, out_ref):
    out_ref[0] = jnp.dot(pooled_ref[0], w0t_ref[...],
                         preferred_element_type=jnp.float32)


def _pool_p0_kern(p0_ref, out_ref):
    x = p0_ref[0]
    c = x.shape[1]
    lane = lax.broadcasted_iota(jnp.int32, x.shape, 1)
    zcol = jnp.zeros((x.shape[0], 1), jnp.float32)
    zblk = jnp.zeros((x.shape[0], HW), jnp.float32)
    left = jnp.concatenate([x[:, 1:], zcol], axis=1)
    left = jnp.where(lane % HW != HW - 1, left, 0.0)
    right = jnp.concatenate([zcol, x[:, :-1]], axis=1)
    right = jnp.where(lane % HW != 0, right, 0.0)
    t = x + left + right
    up = jnp.concatenate([t[:, HW:], zblk], axis=1)
    dn = jnp.concatenate([zblk, t[:, :c - HW]], axis=1)
    out_ref[0] = (t + up + dn) * (1.0 / 9.0)


def _dist_kern(e_ref, cen_ref, cc_ref, out_ref):
    e = e_ref[0, 0]
    feat = jnp.sum(e * e, axis=1, keepdims=True)
    g = jnp.dot(e.astype(jnp.bfloat16), cen_ref[...],
                preferred_element_type=jnp.float32)
    big = jnp.float32(3.0e38)
    one = jnp.float32(1.0)
    eps = jnp.float32(1e-12)
    nfull = NCEN // 128
    ntail = NCEN - nfull * 128
    rows = []
    for rb in range(PXT // RB):
        r0 = rb * RB
        t1 = jnp.full((RB, 128), big, jnp.float32)
        t2 = t1
        t3 = t1
        for c in range(nfull + 1):
            c0 = c * 128
            if c < nfull:
                v = (cc_ref[0:1, c0:c0 + 128]
                     - 2.0 * g[r0:r0 + RB, c0:c0 + 128])
            else:
                v = jnp.concatenate(
                    [cc_ref[0:1, c0:c0 + ntail]
                     - 2.0 * g[r0:r0 + RB, c0:c0 + ntail],
                     jnp.full((RB, 128 - ntail), big, jnp.float32)], axis=1)
            n1 = jnp.minimum(t1, v)
            h1 = jnp.maximum(t1, v)
            n2 = jnp.minimum(t2, h1)
            h2 = jnp.maximum(t2, h1)
            t3 = jnp.minimum(t3, h2)
            t1 = n1
            t2 = n2
        u = jnp.concatenate([t1, t2, t3], axis=1)
        m1 = jnp.min(u, axis=1, keepdims=True)
        eq1 = u == m1
        c1 = jnp.sum(jnp.where(eq1, one, 0.0), axis=1, keepdims=True)
        u2 = jnp.where(eq1, big, u)
        m2r = jnp.min(u2, axis=1, keepdims=True)
        eq2 = u2 == m2r
        c2 = jnp.sum(jnp.where(eq2, one, 0.0), axis=1, keepdims=True)
        m3r = jnp.min(jnp.where(eq2, big, u2), axis=1, keepdims=True)
        m2 = jnp.where(c1 >= 2.0, m1, m2r)
        m3 = jnp.where(c1 >= 3.0, m1,
                       jnp.where(c1 == 2.0, m2r,
                                 jnp.where(c2 >= 2.0, m2r, m3r)))
        fr = feat[r0:r0 + RB]
        d1 = jnp.sqrt(jnp.maximum(fr + m1, eps))
        d2 = jnp.sqrt(jnp.maximum(fr + m2, eps))
        d3 = jnp.sqrt(jnp.maximum(fr + m3, eps))
        rows.append(d1 / (1.0 + jnp.exp(d1 - d2) + jnp.exp(d1 - d3)))
    score = jnp.concatenate(rows, axis=0)
    out_ref[0, 0, 0] = score[:, 0].reshape(1, PXT)[0]


@jax.jit
def kernel(p0, p1, p2, W, bconv, centroids):
    B = p0.shape[0]
    f32 = jnp.float32

    w0t = W[:, :256].T
    w1t = W[:, 256:768].T
    w2t = W[:, 768:1792].T
    wc = jnp.concatenate([W[:, 1792][None], W[:, 1793][None], bconv[None],
                          jnp.zeros((5, CO), f32)], axis=0)

    a1 = jnp.asarray(_A1)
    s2 = jnp.asarray(_S2)
    cm = jnp.asarray(_CM)

    p0v = p0.transpose(0, 1, 3, 2).reshape(B, 256, NPX)
    p1v = p1.reshape(B, 512, 784)
    p2v = p2.reshape(B, 1024, 196)

    pooled0 = pl.pallas_call(
        _pool_p0_kern,
        grid=(B,),
        in_specs=[pl.BlockSpec((1, 256, NPX), lambda b: (b, 0, 0))],
        out_specs=pl.BlockSpec((1, 256, NPX), lambda b: (b, 0, 0)),
        out_shape=jax.ShapeDtypeStruct((B, 256, NPX), f32),
    )(p0v)

    pooled0t = pooled0.transpose(0, 2, 1)

    e0 = pl.pallas_call(
        _mix_p0_kern,
        grid=(B, NT),
        in_specs=[pl.BlockSpec((1, PXT, 256), lambda b, t: (b, t, 0)),
                  pl.BlockSpec((256, CO), lambda b, t: (0, 0))],
        out_specs=pl.BlockSpec((1, PXT, CO), lambda b, t: (b, t, 0)),
        out_shape=jax.ShapeDtypeStruct((B, NPX, CO), f32),
    )(pooled0t, w0t)

    m2 = pl.pallas_call(
        _mix_p2_kern,
        grid=(B,),
        in_specs=[pl.BlockSpec((1, 1024, 196), lambda b: (b, 0, 0)),
                  pl.BlockSpec((1024, CO), lambda b: (0, 0))],
        out_specs=pl.BlockSpec((1, 196, CO), lambda b: (b, 0, 0)),
        out_shape=jax.ShapeDtypeStruct((B, 196, CO), f32),
    )(p2v, w2t)

    m1 = pl.pallas_call(
        _mix_p1_kern,
        grid=(B,),
        in_specs=[pl.BlockSpec((1, 512, 784), lambda b: (b, 0, 0)),
                  pl.BlockSpec((512, CO), lambda b: (0, 0))],
        out_specs=pl.BlockSpec((1, 784, CO), lambda b: (b, 0, 0)),
        out_shape=jax.ShapeDtypeStruct((B, 784, CO), f32),
    )(p1v, w1t)

    m1v = m1.reshape(B, 28, 28 * CO)
    z = pl.pallas_call(
        _hstep_kern,
        grid=(B,),
        in_specs=[pl.BlockSpec((HW, 28), lambda b: (0, 0)),
                  pl.BlockSpec((1, 28, 28 * CO), lambda b: (b, 0, 0))],
        out_specs=pl.BlockSpec((1, HW, 28 * CO), lambda b: (b, 0, 0)),
        out_shape=jax.ShapeDtypeStruct((B, HW, 28 * CO), f32),
    )(a1, m1v)

    zv = z.reshape(B, HW, 28, CO)
    yc = 8
    e1 = pl.pallas_call(
        _wstep_kern,
        grid=(B, HW // yc),
        in_specs=[pl.BlockSpec((HW, 28), lambda b, c: (0, 0)),
                  pl.BlockSpec((1, yc, 28, CO), lambda b, c: (b, c, 0, 0))],
        out_specs=pl.BlockSpec((1, HW, yc, CO), lambda b, c: (b, 0, c, 0)),
        out_shape=jax.ShapeDtypeStruct((B, HW, HW, CO), f32),
    )(a1, zv)

    ee = pl.pallas_call(
        _e2_kern,
        grid=(B, NT),
        in_specs=[pl.BlockSpec((PXT, 196), lambda b, t: (t, 0)),
                  pl.BlockSpec((1, 196, CO), lambda b, t: (b, 0, 0)),
                  pl.BlockSpec((PXT, 8), lambda b, t: (t, 0)),
                  pl.BlockSpec((8, CO), lambda b, t: (0, 0)),
                  pl.BlockSpec((1, PXT, CO), lambda b, t: (b, t, 0)),
                  pl.BlockSpec((1, PXT, CO), lambda b, t: (b, t, 0))],
        out_specs=pl.BlockSpec((1, PXT, CO), lambda b, t: (b, t, 0)),
        out_shape=jax.ShapeDtypeStruct((B, NPX, CO), f32),
    )(s2, m2, cm, wc, e0, e1.reshape(B, NPX, CO))

    cc = pl.pallas_call(
        _cc_kern,
        grid=(8,),
        in_specs=[pl.BlockSpec((CO // 8, NCEN), lambda i: (i, 0))],
        out_specs=pl.BlockSpec((8, NCEN), lambda i: (0, 0)),
        out_shape=jax.ShapeDtypeStruct((8, NCEN), f32),
    )(centroids)

    eev = ee.reshape(B, NT, PXT, CO)

    scores = pl.pallas_call(
        _dist_kern,
        grid=(B, NT),
        in_specs=[pl.BlockSpec((1, 1, PXT, CO), lambda b, t: (b, t, 0, 0)),
                  pl.BlockSpec((CO, NCEN), lambda b, t: (0, 0)),
                  pl.BlockSpec((8, NCEN), lambda b, t: (0, 0))],
        out_specs=pl.BlockSpec((1, 1, 1, PXT), lambda b, t: (b, t, 0, 0)),
        out_shape=jax.ShapeDtypeStruct((B, NT, 1, PXT), f32),
    )(eev, centroids.astype(jnp.bfloat16), cc)

    return scores.reshape(B, HW, HW).transpose(0, 2, 1).reshape(B, 1, HW, HW)


# speedup vs baseline: 1.1302x; 1.1302x over previous
import functools

import numpy as np
import jax
import jax.numpy as jnp
from jax import lax
from jax.experimental import pallas as pl
from jax.experimental.pallas import tpu as pltpu

HW = 56
NPX = HW * HW
CO = 1792
NCEN = 3136
PXT = 224
NT = NPX // PXT
RB = 56


def _pool_mat(n: int) -> np.ndarray:
    m = np.zeros((n, n), np.float32)
    for i in range(n):
        for j in (i - 1, i, i + 1):
            if 0 <= j < n:
                m[i, j] = 1.0 / 3.0
    return m


def _resize_mat(n_out: int, n_in: int) -> np.ndarray:
    m = np.zeros((n_out, n_in), np.float32)
    scale = n_in / n_out
    for i in range(n_out):
        c = (i + 0.5) * scale - 0.5
        j0 = int(np.floor(c))
        f = c - j0
        m[i, min(max(j0, 0), n_in - 1)] += 1.0 - f
        m[i, min(max(j0 + 1, 0), n_in - 1)] += f
    return m


_A1 = (_resize_mat(56, 28) @ _pool_mat(28)).astype(np.float32)
_A2 = (_resize_mat(56, 14) @ _pool_mat(14)).astype(np.float32)
_S2 = np.einsum("yv,xu->xyvu", _A2, _A2).reshape(NPX, 196).astype(np.float32)

_xx = (np.arange(56, dtype=np.float32) / 55.0) * 2.0 - 1.0
_CM = np.zeros((NPX, 8), np.float32)
_CM[:, 0] = np.repeat(_xx, 56)
_CM[:, 1] = np.tile(_xx, 56)
_CM[:, 2] = 1.0


def _mix_p2_kern(p2_ref, w2t_ref, out_ref):
    out_ref[0] = lax.dot_general(p2_ref[0], w2t_ref[...],
                                 (((0,), (0,)), ((), ())),
                                 preferred_element_type=jnp.float32)


def _e2_kern(s2_ref, m2_ref, cm_ref, wc_ref, pool_ref, w0t_ref, e1_ref, out_ref):
    e2 = jnp.dot(s2_ref[...], m2_ref[0], preferred_element_type=jnp.float32)
    e2 = e2 + jnp.dot(cm_ref[...], wc_ref[...], preferred_element_type=jnp.float32)
    e2 = e2 + jnp.dot(pool_ref[0], w0t_ref[...], preferred_element_type=jnp.float32)
    out_ref[0] = e2 + e1_ref[0]


def _cc_kern(cen_ref, out_ref):
    i = pl.program_id(0)
    part = jnp.broadcast_to(
        jnp.sum(cen_ref[...] * cen_ref[...], axis=0, keepdims=True),
        out_ref.shape)

    @pl.when(i == 0)
    def _():
        out_ref[...] = part

    @pl.when(i > 0)
    def _():
        out_ref[...] = out_ref[...] + part


def _mix_p1_kern(p1_ref, w1t_ref, out_ref):
    out_ref[0] = lax.dot_general(p1_ref[0], w1t_ref[...],
                                 (((0,), (0,)), ((), ())),
                                 preferred_element_type=jnp.float32)


def _hstep_kern(a_ref, m1_ref, out_ref):
    out_ref[0] = jnp.dot(a_ref[...], m1_ref[0], preferred_element_type=jnp.float32)


def _wstep_kern(a_ref, z_ref, out_ref):
    for j in range(z_ref.shape[1]):
        out_ref[0, :, j, :] = jnp.dot(a_ref[...], z_ref[0, j],
                                      preferred_element_type=jnp.float32)


def _mix_p0_kern(pooled_ref, w0t_ref, out_ref):
    out_ref[0] = jnp.dot(pooled_ref[0], w0t_ref[...],
                         preferred_element_type=jnp.float32)


def _pool_p0_kern(p0_ref, out_ref):
    x = p0_ref[0]
    c = x.shape[1]
    lane = lax.broadcasted_iota(jnp.int32, x.shape, 1)
    zcol = jnp.zeros((x.shape[0], 1), jnp.float32)
    zblk = jnp.zeros((x.shape[0], HW), jnp.float32)
    left = jnp.concatenate([x[:, 1:], zcol], axis=1)
    left = jnp.where(lane % HW != HW - 1, left, 0.0)
    right = jnp.concatenate([zcol, x[:, :-1]], axis=1)
    right = jnp.where(lane % HW != 0, right, 0.0)
    t = x + left + right
    up = jnp.concatenate([t[:, HW:], zblk], axis=1)
    dn = jnp.concatenate([zblk, t[:, :c - HW]], axis=1)
    out_ref[0] = (t + up + dn) * (1.0 / 9.0)


def _dist_kern(e_ref, cen_ref, cc_ref, out_ref):
    e = e_ref[0, 0]
    feat = jnp.sum(e * e, axis=1, keepdims=True)
    g = jnp.dot(e, cen_ref[...], preferred_element_type=jnp.float32)
    big = jnp.float32(3.0e38)
    one = jnp.float32(1.0)
    eps = jnp.float32(1e-12)
    nfull = NCEN // 128
    ntail = NCEN - nfull * 128
    rows = []
    for rb in range(PXT // RB):
        r0 = rb * RB
        t1 = jnp.full((RB, 128), big, jnp.float32)
        t2 = t1
        t3 = t1
        for c in range(nfull + 1):
            c0 = c * 128
            if c < nfull:
                v = (cc_ref[0:1, c0:c0 + 128]
                     - 2.0 * g[r0:r0 + RB, c0:c0 + 128])
            else:
                v = jnp.concatenate(
                    [cc_ref[0:1, c0:c0 + ntail]
                     - 2.0 * g[r0:r0 + RB, c0:c0 + ntail],
                     jnp.full((RB, 128 - ntail), big, jnp.float32)], axis=1)
            n1 = jnp.minimum(t1, v)
            h1 = jnp.maximum(t1, v)
            n2 = jnp.minimum(t2, h1)
            h2 = jnp.maximum(t2, h1)
            t3 = jnp.minimum(t3, h2)
            t1 = n1
            t2 = n2
        u = jnp.concatenate([t1, t2, t3], axis=1)
        m1 = jnp.min(u, axis=1, keepdims=True)
        eq1 = u == m1
        c1 = jnp.sum(jnp.where(eq1, one, 0.0), axis=1, keepdims=True)
        u2 = jnp.where(eq1, big, u)
        m2r = jnp.min(u2, axis=1, keepdims=True)
        eq2 = u2 == m2r
        c2 = jnp.sum(jnp.where(eq2, one, 0.0), axis=1, keepdims=True)
        m3r = jnp.min(jnp.where(eq2, big, u2), axis=1, keepdims=True)
        m2 = jnp.where(c1 >= 2.0, m1, m2r)
        m3 = jnp.where(c1 >= 3.0, m1,
                       jnp.where(c1 == 2.0, m2r,
                                 jnp.where(c2 >= 2.0, m2r, m3r)))
        fr = feat[r0:r0 + RB]
        d1 = jnp.sqrt(jnp.maximum(fr + m1, eps))
        d2 = jnp.sqrt(jnp.maximum(fr + m2, eps))
        d3 = jnp.sqrt(jnp.maximum(fr + m3, eps))
        rows.append(d1 / (1.0 + jnp.exp(d1 - d2) + jnp.exp(d1 - d3)))
    score = jnp.concatenate(rows, axis=0)
    out_ref[0, 0, 0] = score[:, 0].reshape(1, PXT)[0]


@jax.jit
def kernel(p0, p1, p2, W, bconv, centroids):
    B = p0.shape[0]
    f32 = jnp.float32

    w0t = W[:, :256].T
    w1t = W[:, 256:768].T
    w2t = W[:, 768:1792].T
    wc = jnp.concatenate([W[:, 1792][None], W[:, 1793][None], bconv[None],
                          jnp.zeros((5, CO), f32)], axis=0)

    a1 = jnp.asarray(_A1)
    s2 = jnp.asarray(_S2)
    cm = jnp.asarray(_CM)

    p0v = p0.transpose(0, 1, 3, 2).reshape(B, 256, NPX)
    p1v = p1.reshape(B, 512, 784)
    p2v = p2.reshape(B, 1024, 196)

    pooled0 = pl.pallas_call(
        _pool_p0_kern,
        grid=(B,),
        in_specs=[pl.BlockSpec((1, 256, NPX), lambda b: (b, 0, 0))],
        out_specs=pl.BlockSpec((1, 256, NPX), lambda b: (b, 0, 0)),
        out_shape=jax.ShapeDtypeStruct((B, 256, NPX), f32),
    )(p0v)

    pooled0t = pooled0.transpose(0, 2, 1)

    m2 = pl.pallas_call(
        _mix_p2_kern,
        grid=(B,),
        in_specs=[pl.BlockSpec((1, 1024, 196), lambda b: (b, 0, 0)),
                  pl.BlockSpec((1024, CO), lambda b: (0, 0))],
        out_specs=pl.BlockSpec((1, 196, CO), lambda b: (b, 0, 0)),
        out_shape=jax.ShapeDtypeStruct((B, 196, CO), f32),
    )(p2v, w2t)

    m1 = pl.pallas_call(
        _mix_p1_kern,
        grid=(B,),
        in_specs=[pl.BlockSpec((1, 512, 784), lambda b: (b, 0, 0)),
                  pl.BlockSpec((512, CO), lambda b: (0, 0))],
        out_specs=pl.BlockSpec((1, 784, CO), lambda b: (b, 0, 0)),
        out_shape=jax.ShapeDtypeStruct((B, 784, CO), f32),
    )(p1v, w1t)

    m1v = m1.reshape(B, 28, 28 * CO)
    z = pl.pallas_call(
        _hstep_kern,
        grid=(B,),
        in_specs=[pl.BlockSpec((HW, 28), lambda b: (0, 0)),
                  pl.BlockSpec((1, 28, 28 * CO), lambda b: (b, 0, 0))],
        out_specs=pl.BlockSpec((1, HW, 28 * CO), lambda b: (b, 0, 0)),
        out_shape=jax.ShapeDtypeStruct((B, HW, 28 * CO), f32),
    )(a1, m1v)

    zv = z.reshape(B, HW, 28, CO)
    yc = 8
    e1 = pl.pallas_call(
        _wstep_kern,
        grid=(B, HW // yc),
        in_specs=[pl.BlockSpec((HW, 28), lambda b, c: (0, 0)),
                  pl.BlockSpec((1, yc, 28, CO), lambda b, c: (b, c, 0, 0))],
        out_specs=pl.BlockSpec((1, HW, yc, CO), lambda b, c: (b, 0, c, 0)),
        out_shape=jax.ShapeDtypeStruct((B, HW, HW, CO), f32),
    )(a1, zv)

    ee = pl.pallas_call(
        _e2_kern,
        grid=(B, NT),
        in_specs=[pl.BlockSpec((PXT, 196), lambda b, t: (t, 0)),
                  pl.BlockSpec((1, 196, CO), lambda b, t: (b, 0, 0)),
                  pl.BlockSpec((PXT, 8), lambda b, t: (t, 0)),
                  pl.BlockSpec((8, CO), lambda b, t: (0, 0)),
                  pl.BlockSpec((1, PXT, 256), lambda b, t: (b, t, 0)),
                  pl.BlockSpec((256, CO), lambda b, t: (0, 0)),
                  pl.BlockSpec((1, PXT, CO), lambda b, t: (b, t, 0))],
        out_specs=pl.BlockSpec((1, PXT, CO), lambda b, t: (b, t, 0)),
        out_shape=jax.ShapeDtypeStruct((B, NPX, CO), f32),
    )(s2, m2, cm, wc, pooled0t, w0t, e1.reshape(B, NPX, CO))

    cc = pl.pallas_call(
        _cc_kern,
        grid=(8,),
        in_specs=[pl.BlockSpec((CO // 8, NCEN), lambda i: (i, 0))],
        out_specs=pl.BlockSpec((8, NCEN), lambda i: (0, 0)),
        out_shape=jax.ShapeDtypeStruct((8, NCEN), f32),
    )(centroids)

    eev = ee.reshape(B, NT, PXT, CO)

    scores = pl.pallas_call(
        _dist_kern,
        grid=(B, NT),
        in_specs=[pl.BlockSpec((1, 1, PXT, CO), lambda b, t: (b, t, 0, 0)),
                  pl.BlockSpec((CO, NCEN), lambda b, t: (0, 0)),
                  pl.BlockSpec((8, NCEN), lambda b, t: (0, 0))],
        out_specs=pl.BlockSpec((1, 1, 1, PXT), lambda b, t: (b, t, 0, 0)),
        out_shape=jax.ShapeDtypeStruct((B, NT, 1, PXT), f32),
    )(eev, centroids, cc)

    return scores.reshape(B, HW, HW).transpose(0, 2, 1).reshape(B, 1, HW, HW)


# speedup vs baseline: 1.1892x; 1.0522x over previous
import functools

import numpy as np
import jax
import jax.numpy as jnp
from jax import lax
from jax.experimental import pallas as pl
from jax.experimental.pallas import tpu as pltpu

HW = 56
NPX = HW * HW
CO = 1792
NCEN = 3136
PXT = 448
NT = NPX // PXT
RB = 56


def _pool_mat(n: int) -> np.ndarray:
    m = np.zeros((n, n), np.float32)
    for i in range(n):
        for j in (i - 1, i, i + 1):
            if 0 <= j < n:
                m[i, j] = 1.0 / 3.0
    return m


def _resize_mat(n_out: int, n_in: int) -> np.ndarray:
    m = np.zeros((n_out, n_in), np.float32)
    scale = n_in / n_out
    for i in range(n_out):
        c = (i + 0.5) * scale - 0.5
        j0 = int(np.floor(c))
        f = c - j0
        m[i, min(max(j0, 0), n_in - 1)] += 1.0 - f
        m[i, min(max(j0 + 1, 0), n_in - 1)] += f
    return m


_A1 = (_resize_mat(56, 28) @ _pool_mat(28)).astype(np.float32)
_A2 = (_resize_mat(56, 14) @ _pool_mat(14)).astype(np.float32)
_S2 = np.einsum("yv,xu->xyvu", _A2, _A2).reshape(NPX, 196).astype(np.float32)

_xx = (np.arange(56, dtype=np.float32) / 55.0) * 2.0 - 1.0
_CM = np.zeros((NPX, 8), np.float32)
_CM[:, 0] = np.repeat(_xx, 56)
_CM[:, 1] = np.tile(_xx, 56)
_CM[:, 2] = 1.0


def _mix_p2_kern(p2_ref, w2t_ref, out_ref):
    out_ref[0] = lax.dot_general(p2_ref[0], w2t_ref[...],
                                 (((0,), (0,)), ((), ())),
                                 preferred_element_type=jnp.float32)


def _e2_kern(s2_ref, m2_ref, cm_ref, wc_ref, pool_ref, w0t_ref, e1_ref, out_ref):
    e2 = jnp.dot(s2_ref[...], m2_ref[0], preferred_element_type=jnp.float32)
    e2 = e2 + jnp.dot(cm_ref[...], wc_ref[...], preferred_element_type=jnp.float32)
    e2 = e2 + jnp.dot(pool_ref[0], w0t_ref[...], preferred_element_type=jnp.float32)
    out_ref[0] = e2 + e1_ref[0]


def _cc_kern(cen_ref, out_ref):
    i = pl.program_id(0)
    part = jnp.broadcast_to(
        jnp.sum(cen_ref[...] * cen_ref[...], axis=0, keepdims=True),
        out_ref.shape)

    @pl.when(i == 0)
    def _():
        out_ref[...] = part

    @pl.when(i > 0)
    def _():
        out_ref[...] = out_ref[...] + part


def _mix_p1_kern(p1_ref, w1t_ref, out_ref):
    out_ref[0] = lax.dot_general(p1_ref[0], w1t_ref[...],
                                 (((0,), (0,)), ((), ())),
                                 preferred_element_type=jnp.float32)


def _hstep_kern(a_ref, m1_ref, out_ref):
    out_ref[0] = jnp.dot(a_ref[...], m1_ref[0], preferred_element_type=jnp.float32)


def _wstep_kern(a_ref, z_ref, out_ref):
    for j in range(z_ref.shape[1]):
        out_ref[0, :, j, :] = jnp.dot(a_ref[...], z_ref[0, j],
                                      preferred_element_type=jnp.float32)


def _mix_p0_kern(pooled_ref, w0t_ref, out_ref):
    out_ref[0] = jnp.dot(pooled_ref[0], w0t_ref[...],
                         preferred_element_type=jnp.float32)


def _pool_p0_kern(p0_ref, out_ref):
    x = p0_ref[0]
    c = x.shape[1]
    lane = lax.broadcasted_iota(jnp.int32, x.shape, 1)
    zcol = jnp.zeros((x.shape[0], 1), jnp.float32)
    zblk = jnp.zeros((x.shape[0], HW), jnp.float32)
    left = jnp.concatenate([x[:, 1:], zcol], axis=1)
    left = jnp.where(lane % HW != HW - 1, left, 0.0)
    right = jnp.concatenate([zcol, x[:, :-1]], axis=1)
    right = jnp.where(lane % HW != 0, right, 0.0)
    t = x + left + right
    up = jnp.concatenate([t[:, HW:], zblk], axis=1)
    dn = jnp.concatenate([zblk, t[:, :c - HW]], axis=1)
    out_ref[0] = (t + up + dn) * (1.0 / 9.0)


def _dist_kern(e_ref, cen_ref, cc_ref, out_ref):
    e = e_ref[0, 0]
    feat = jnp.sum(e * e, axis=1, keepdims=True)
    g = jnp.dot(e, cen_ref[...], preferred_element_type=jnp.float32)
    big = jnp.float32(3.0e38)
    one = jnp.float32(1.0)
    eps = jnp.float32(1e-12)
    nfull = NCEN // 128
    ntail = NCEN - nfull * 128
    rows = []
    for rb in range(PXT // RB):
        r0 = rb * RB
        t1 = jnp.full((RB, 128), big, jnp.float32)
        t2 = t1
        t3 = t1
        for c in range(nfull + 1):
            c0 = c * 128
            if c < nfull:
                v = (cc_ref[0:1, c0:c0 + 128]
                     - 2.0 * g[r0:r0 + RB, c0:c0 + 128])
            else:
                v = jnp.concatenate(
                    [cc_ref[0:1, c0:c0 + ntail]
                     - 2.0 * g[r0:r0 + RB, c0:c0 + ntail],
                     jnp.full((RB, 128 - ntail), big, jnp.float32)], axis=1)
            n1 = jnp.minimum(t1, v)
            h1 = jnp.maximum(t1, v)
            n2 = jnp.minimum(t2, h1)
            h2 = jnp.maximum(t2, h1)
            t3 = jnp.minimum(t3, h2)
            t1 = n1
            t2 = n2
        u = jnp.concatenate([t1, t2, t3], axis=1)
        m1 = jnp.min(u, axis=1, keepdims=True)
        eq1 = u == m1
        c1 = jnp.sum(jnp.where(eq1, one, 0.0), axis=1, keepdims=True)
        u2 = jnp.where(eq1, big, u)
        m2r = jnp.min(u2, axis=1, keepdims=True)
        eq2 = u2 == m2r
        c2 = jnp.sum(jnp.where(eq2, one, 0.0), axis=1, keepdims=True)
        m3r = jnp.min(jnp.where(eq2, big, u2), axis=1, keepdims=True)
        m2 = jnp.where(c1 >= 2.0, m1, m2r)
        m3 = jnp.where(c1 >= 3.0, m1,
                       jnp.where(c1 == 2.0, m2r,
                                 jnp.where(c2 >= 2.0, m2r, m3r)))
        fr = feat[r0:r0 + RB]
        d1 = jnp.sqrt(jnp.maximum(fr + m1, eps))
        d2 = jnp.sqrt(jnp.maximum(fr + m2, eps))
        d3 = jnp.sqrt(jnp.maximum(fr + m3, eps))
        rows.append(d1 / (1.0 + jnp.exp(d1 - d2) + jnp.exp(d1 - d3)))
    score = jnp.concatenate(rows, axis=0)
    out_ref[0, 0, 0] = score[:, 0].reshape(1, PXT)[0]


@jax.jit
def kernel(p0, p1, p2, W, bconv, centroids):
    B = p0.shape[0]
    f32 = jnp.float32

    w0t = W[:, :256].T
    w1t = W[:, 256:768].T
    w2t = W[:, 768:1792].T
    wc = jnp.concatenate([W[:, 1792][None], W[:, 1793][None], bconv[None],
                          jnp.zeros((5, CO), f32)], axis=0)

    a1 = jnp.asarray(_A1)
    s2 = jnp.asarray(_S2)
    cm = jnp.asarray(_CM)

    p0v = p0.transpose(0, 1, 3, 2).reshape(B, 256, NPX)
    p1v = p1.reshape(B, 512, 784)
    p2v = p2.reshape(B, 1024, 196)

    pooled0 = pl.pallas_call(
        _pool_p0_kern,
        grid=(B,),
        in_specs=[pl.BlockSpec((1, 256, NPX), lambda b: (b, 0, 0))],
        out_specs=pl.BlockSpec((1, 256, NPX), lambda b: (b, 0, 0)),
        out_shape=jax.ShapeDtypeStruct((B, 256, NPX), f32),
    )(p0v)

    pooled0t = pooled0.transpose(0, 2, 1)

    m2 = pl.pallas_call(
        _mix_p2_kern,
        grid=(B,),
        in_specs=[pl.BlockSpec((1, 1024, 196), lambda b: (b, 0, 0)),
                  pl.BlockSpec((1024, CO), lambda b: (0, 0))],
        out_specs=pl.BlockSpec((1, 196, CO), lambda b: (b, 0, 0)),
        out_shape=jax.ShapeDtypeStruct((B, 196, CO), f32),
    )(p2v, w2t)

    m1 = pl.pallas_call(
        _mix_p1_kern,
        grid=(B,),
        in_specs=[pl.BlockSpec((1, 512, 784), lambda b: (b, 0, 0)),
                  pl.BlockSpec((512, CO), lambda b: (0, 0))],
        out_specs=pl.BlockSpec((1, 784, CO), lambda b: (b, 0, 0)),
        out_shape=jax.ShapeDtypeStruct((B, 784, CO), f32),
    )(p1v, w1t)

    m1v = m1.reshape(B, 28, 28 * CO)
    z = pl.pallas_call(
        _hstep_kern,
        grid=(B,),
        in_specs=[pl.BlockSpec((HW, 28), lambda b: (0, 0)),
                  pl.BlockSpec((1, 28, 28 * CO), lambda b: (b, 0, 0))],
        out_specs=pl.BlockSpec((1, HW, 28 * CO), lambda b: (b, 0, 0)),
        out_shape=jax.ShapeDtypeStruct((B, HW, 28 * CO), f32),
    )(a1, m1v)

    zv = z.reshape(B, HW, 28, CO)
    yc = 8
    e1 = pl.pallas_call(
        _wstep_kern,
        grid=(B, HW // yc),
        in_specs=[pl.BlockSpec((HW, 28), lambda b, c: (0, 0)),
                  pl.BlockSpec((1, yc, 28, CO), lambda b, c: (b, c, 0, 0))],
        out_specs=pl.BlockSpec((1, HW, yc, CO), lambda b, c: (b, 0, c, 0)),
        out_shape=jax.ShapeDtypeStruct((B, HW, HW, CO), f32),
    )(a1, zv)

    ee = pl.pallas_call(
        _e2_kern,
        grid=(B, NT),
        in_specs=[pl.BlockSpec((PXT, 196), lambda b, t: (t, 0)),
                  pl.BlockSpec((1, 196, CO), lambda b, t: (b, 0, 0)),
                  pl.BlockSpec((PXT, 8), lambda b, t: (t, 0)),
                  pl.BlockSpec((8, CO), lambda b, t: (0, 0)),
                  pl.BlockSpec((1, PXT, 256), lambda b, t: (b, t, 0)),
                  pl.BlockSpec((256, CO), lambda b, t: (0, 0)),
                  pl.BlockSpec((1, PXT, CO), lambda b, t: (b, t, 0))],
        out_specs=pl.BlockSpec((1, PXT, CO), lambda b, t: (b, t, 0)),
        out_shape=jax.ShapeDtypeStruct((B, NPX, CO), f32),
    )(s2, m2, cm, wc, pooled0t, w0t, e1.reshape(B, NPX, CO))

    cc = pl.pallas_call(
        _cc_kern,
        grid=(8,),
        in_specs=[pl.BlockSpec((CO // 8, NCEN), lambda i: (i, 0))],
        out_specs=pl.BlockSpec((8, NCEN), lambda i: (0, 0)),
        out_shape=jax.ShapeDtypeStruct((8, NCEN), f32),
    )(centroids)

    eev = ee.reshape(B, NT, PXT, CO)

    scores = pl.pallas_call(
        _dist_kern,
        grid=(B, NT),
        in_specs=[pl.BlockSpec((1, 1, PXT, CO), lambda b, t: (b, t, 0, 0)),
                  pl.BlockSpec((CO, NCEN), lambda b, t: (0, 0)),
                  pl.BlockSpec((8, NCEN), lambda b, t: (0, 0))],
        out_specs=pl.BlockSpec((1, 1, 1, PXT), lambda b, t: (b, t, 0, 0)),
        out_shape=jax.ShapeDtypeStruct((B, NT, 1, PXT), f32),
    )(eev, centroids, cc)

    return scores.reshape(B, HW, HW).transpose(0, 2, 1).reshape(B, 1, HW, HW)


# speedup vs baseline: 1.1892x; 1.0001x over previous
import numpy as np
import jax
import jax.numpy as jnp
from jax import lax
from jax.experimental import pallas as pl
from jax.experimental.pallas import tpu as pltpu

HW = 56
NPX = HW * HW
CO = 1792
NCEN = 3136
PXT = 448
NT = NPX // PXT
RB = 56


def _pool_mat(n: int) -> np.ndarray:
    m = np.zeros((n, n), np.float32)
    for i in range(n):
        for j in (i - 1, i, i + 1):
            if 0 <= j < n:
                m[i, j] = 1.0 / 3.0
    return m


def _resize_mat(n_out: int, n_in: int) -> np.ndarray:
    m = np.zeros((n_out, n_in), np.float32)
    scale = n_in / n_out
    for i in range(n_out):
        c = (i + 0.5) * scale - 0.5
        j0 = int(np.floor(c))
        f = c - j0
        m[i, min(max(j0, 0), n_in - 1)] += 1.0 - f
        m[i, min(max(j0 + 1, 0), n_in - 1)] += f
    return m


_A1 = (_resize_mat(56, 28) @ _pool_mat(28)).astype(np.float32)
_A2 = (_resize_mat(56, 14) @ _pool_mat(14)).astype(np.float32)
_S2 = np.einsum("yv,xu->xyvu", _A2, _A2).reshape(NPX, 196).astype(np.float32)

_xx = (np.arange(56, dtype=np.float32) / 55.0) * 2.0 - 1.0
_CM = np.zeros((NPX, 8), np.float32)
_CM[:, 0] = np.repeat(_xx, 56)
_CM[:, 1] = np.tile(_xx, 56)
_CM[:, 2] = 1.0


def _mix_p2_kern(p2_ref, w2t_ref, out_ref):
    out_ref[0] = lax.dot_general(p2_ref[0], w2t_ref[...],
                                 (((0,), (0,)), ((), ())),
                                 preferred_element_type=jnp.float32)


def _e2_kern(s2_ref, m2_ref, cm_ref, wc_ref, pool_ref, w0t_ref, e1_ref, out_ref):
    e2 = jnp.dot(s2_ref[...], m2_ref[0], preferred_element_type=jnp.float32)
    e2 = e2 + jnp.dot(cm_ref[...], wc_ref[...], preferred_element_type=jnp.float32)
    e2 = e2 + jnp.dot(pool_ref[0], w0t_ref[...], preferred_element_type=jnp.float32)
    out_ref[0] = e2 + e1_ref[0]


def _cc_kern(cen_ref, out_ref):
    i = pl.program_id(0)
    part = jnp.broadcast_to(
        jnp.sum(cen_ref[...] * cen_ref[...], axis=0, keepdims=True),
        out_ref.shape)

    @pl.when(i == 0)
    def _():
        out_ref[...] = part

    @pl.when(i > 0)
    def _():
        out_ref[...] = out_ref[...] + part


def _mix_p1_kern(p1_ref, w1t_ref, out_ref):
    out_ref[0] = lax.dot_general(p1_ref[0], w1t_ref[...],
                                 (((0,), (0,)), ((), ())),
                                 preferred_element_type=jnp.float32)


def _hstep_kern(a_ref, m1_ref, out_ref):
    out_ref[0] = jnp.dot(a_ref[...], m1_ref[0], preferred_element_type=jnp.float32)


def _wstep_kern(a_ref, z_ref, out_ref):
    for j in range(z_ref.shape[1]):
        out_ref[0, :, j, :] = jnp.dot(a_ref[...], z_ref[0, j],
                                      preferred_element_type=jnp.float32)


def _pool_p0_kern(p0_ref, out_ref):
    x = p0_ref[0]
    c = x.shape[1]
    lane = lax.broadcasted_iota(jnp.int32, x.shape, 1)
    zcol = jnp.zeros((x.shape[0], 1), jnp.float32)
    zblk = jnp.zeros((x.shape[0], HW), jnp.float32)
    left = jnp.concatenate([x[:, 1:], zcol], axis=1)
    left = jnp.where(lane % HW != HW - 1, left, 0.0)
    right = jnp.concatenate([zcol, x[:, :-1]], axis=1)
    right = jnp.where(lane % HW != 0, right, 0.0)
    t = x + left + right
    up = jnp.concatenate([t[:, HW:], zblk], axis=1)
    dn = jnp.concatenate([zblk, t[:, :c - HW]], axis=1)
    out_ref[0] = (t + up + dn) * (1.0 / 9.0)


def _dist_kern(e_ref, cen_ref, cc_ref, out_ref):
    e = e_ref[0, 0]
    feat = jnp.sum(e * e, axis=1, keepdims=True)
    g = jnp.dot(e, cen_ref[...], preferred_element_type=jnp.float32)
    big = jnp.float32(3.0e38)
    one = jnp.float32(1.0)
    eps = jnp.float32(1e-12)
    nfull = NCEN // 128
    ntail = NCEN - nfull * 128
    rows = []
    for rb in range(PXT // RB):
        r0 = rb * RB
        t1 = jnp.full((RB, 128), big, jnp.float32)
        t2 = t1
        t3 = t1
        for c in range(nfull + 1):
            c0 = c * 128
            if c < nfull:
                v = (cc_ref[0:1, c0:c0 + 128]
                     - 2.0 * g[r0:r0 + RB, c0:c0 + 128])
            else:
                v = jnp.concatenate(
                    [cc_ref[0:1, c0:c0 + ntail]
                     - 2.0 * g[r0:r0 + RB, c0:c0 + ntail],
                     jnp.full((RB, 128 - ntail), big, jnp.float32)], axis=1)
            n1 = jnp.minimum(t1, v)
            h1 = jnp.maximum(t1, v)
            n2 = jnp.minimum(t2, h1)
            h2 = jnp.maximum(t2, h1)
            t3 = jnp.minimum(t3, h2)
            t1 = n1
            t2 = n2
        u = jnp.concatenate([t1, t2, t3], axis=1)
        m1 = jnp.min(u, axis=1, keepdims=True)
        eq1 = u == m1
        c1 = jnp.sum(jnp.where(eq1, one, 0.0), axis=1, keepdims=True)
        u2 = jnp.where(eq1, big, u)
        m2r = jnp.min(u2, axis=1, keepdims=True)
        eq2 = u2 == m2r
        c2 = jnp.sum(jnp.where(eq2, one, 0.0), axis=1, keepdims=True)
        m3r = jnp.min(jnp.where(eq2, big, u2), axis=1, keepdims=True)
        m2 = jnp.where(c1 >= 2.0, m1, m2r)
        m3 = jnp.where(c1 >= 3.0, m1,
                       jnp.where(c1 == 2.0, m2r,
                                 jnp.where(c2 >= 2.0, m2r, m3r)))
        fr = feat[r0:r0 + RB]
        d1 = jnp.sqrt(jnp.maximum(fr + m1, eps))
        d2 = jnp.sqrt(jnp.maximum(fr + m2, eps))
        d3 = jnp.sqrt(jnp.maximum(fr + m3, eps))
        rows.append(d1 / (1.0 + jnp.exp(d1 - d2) + jnp.exp(d1 - d3)))
    score = jnp.concatenate(rows, axis=0)
    out_ref[0, 0, 0] = score[:, 0].reshape(1, PXT)[0]


@jax.jit
def kernel(p0, p1, p2, W, bconv, centroids):
    B = p0.shape[0]
    f32 = jnp.float32

    w0t = W[:, :256].T
    w1t = W[:, 256:768].T
    w2t = W[:, 768:1792].T
    wc = jnp.concatenate([W[:, 1792][None], W[:, 1793][None], bconv[None],
                          jnp.zeros((5, CO), f32)], axis=0)

    a1 = jnp.asarray(_A1)
    s2 = jnp.asarray(_S2)
    cm = jnp.asarray(_CM)

    p0v = p0.transpose(0, 1, 3, 2).reshape(B, 256, NPX)
    p1v = p1.reshape(B, 512, 784)
    p2v = p2.reshape(B, 1024, 196)

    pooled0 = pl.pallas_call(
        _pool_p0_kern,
        grid=(B,),
        in_specs=[pl.BlockSpec((1, 256, NPX), lambda b: (b, 0, 0))],
        out_specs=pl.BlockSpec((1, 256, NPX), lambda b: (b, 0, 0)),
        out_shape=jax.ShapeDtypeStruct((B, 256, NPX), f32),
    )(p0v)

    pooled0t = pooled0.transpose(0, 2, 1)

    m2 = pl.pallas_call(
        _mix_p2_kern,
        grid=(B,),
        in_specs=[pl.BlockSpec((1, 1024, 196), lambda b: (b, 0, 0)),
                  pl.BlockSpec((1024, CO), lambda b: (0, 0))],
        out_specs=pl.BlockSpec((1, 196, CO), lambda b: (b, 0, 0)),
        out_shape=jax.ShapeDtypeStruct((B, 196, CO), f32),
    )(p2v, w2t)

    m1 = pl.pallas_call(
        _mix_p1_kern,
        grid=(B,),
        in_specs=[pl.BlockSpec((1, 512, 784), lambda b: (b, 0, 0)),
                  pl.BlockSpec((512, CO), lambda b: (0, 0))],
        out_specs=pl.BlockSpec((1, 784, CO), lambda b: (b, 0, 0)),
        out_shape=jax.ShapeDtypeStruct((B, 784, CO), f32),
    )(p1v, w1t)

    m1v = m1.reshape(B, 28, 28 * CO)
    z = pl.pallas_call(
        _hstep_kern,
        grid=(B,),
        in_specs=[pl.BlockSpec((HW, 28), lambda b: (0, 0)),
                  pl.BlockSpec((1, 28, 28 * CO), lambda b: (b, 0, 0))],
        out_specs=pl.BlockSpec((1, HW, 28 * CO), lambda b: (b, 0, 0)),
        out_shape=jax.ShapeDtypeStruct((B, HW, 28 * CO), f32),
    )(a1, m1v)

    zv = z.reshape(B, HW, 28, CO)
    yc = 8
    e1 = pl.pallas_call(
        _wstep_kern,
        grid=(B, HW // yc),
        in_specs=[pl.BlockSpec((HW, 28), lambda b, c: (0, 0)),
                  pl.BlockSpec((1, yc, 28, CO), lambda b, c: (b, c, 0, 0))],
        out_specs=pl.BlockSpec((1, HW, yc, CO), lambda b, c: (b, 0, c, 0)),
        out_shape=jax.ShapeDtypeStruct((B, HW, HW, CO), f32),
    )(a1, zv)

    ee = pl.pallas_call(
        _e2_kern,
        grid=(B, NT),
        in_specs=[pl.BlockSpec((PXT, 196), lambda b, t: (t, 0)),
                  pl.BlockSpec((1, 196, CO), lambda b, t: (b, 0, 0)),
                  pl.BlockSpec((PXT, 8), lambda b, t: (t, 0)),
                  pl.BlockSpec((8, CO), lambda b, t: (0, 0)),
                  pl.BlockSpec((1, PXT, 256), lambda b, t: (b, t, 0)),
                  pl.BlockSpec((256, CO), lambda b, t: (0, 0)),
                  pl.BlockSpec((1, PXT, CO), lambda b, t: (b, t, 0))],
        out_specs=pl.BlockSpec((1, PXT, CO), lambda b, t: (b, t, 0)),
        out_shape=jax.ShapeDtypeStruct((B, NPX, CO), f32),
    )(s2, m2, cm, wc, pooled0t, w0t, e1.reshape(B, NPX, CO))

    cc = pl.pallas_call(
        _cc_kern,
        grid=(8,),
        in_specs=[pl.BlockSpec((CO // 8, NCEN), lambda i: (i, 0))],
        out_specs=pl.BlockSpec((8, NCEN), lambda i: (0, 0)),
        out_shape=jax.ShapeDtypeStruct((8, NCEN), f32),
    )(centroids)

    eev = ee.reshape(B, NT, PXT, CO)

    scores = pl.pallas_call(
        _dist_kern,
        grid=(B, NT),
        in_specs=[pl.BlockSpec((1, 1, PXT, CO), lambda b, t: (b, t, 0, 0)),
                  pl.BlockSpec((CO, NCEN), lambda b, t: (0, 0)),
                  pl.BlockSpec((8, NCEN), lambda b, t: (0, 0))],
        out_specs=pl.BlockSpec((1, 1, 1, PXT), lambda b, t: (b, t, 0, 0)),
        out_shape=jax.ShapeDtypeStruct((B, NT, 1, PXT), f32),
    )(eev, centroids, cc)

    return scores.reshape(B, HW, HW).transpose(0, 2, 1).reshape(B, 1, HW, HW)
